# baseline (speedup 1.0000x reference)
import functools

import jax
import jax.numpy as jnp
from jax import lax
from jax.experimental import pallas as pl
from jax.experimental.pallas import tpu as pltpu
from jax.experimental.pallas import tpu_sc as plsc

VOCAB_ = 100000
EMBED_ = 128
HIDDEN_ = 1024
OUT_ = 2
SEQ_ = 20
BATCH_ = 16384
EPS_ = 1e-5

_NC = 2
_NS = 16
_NW = _NC * _NS

_NSL = 2
_BH = BATCH_ // _NSL
_BPW = _BH // _NW
_CH = 64
_NCH = _BPW // _CH


def _pool_body(text_hbm, table_hbm, out_hbm, idx_v, acc_v, sems):
    wid = lax.axis_index("s") * _NC + lax.axis_index("c")
    base = wid * _BPW
    pltpu.sync_copy(text_hbm.at[:, pl.ds(base, _BPW)], idx_v)
    for s in range(SEQ_):
        cps = [pltpu.async_copy(
                   table_hbm.at[idx_v.at[s, pl.ds(c * _CH, _CH)]],
                   acc_v.at[c], sems.at[c], add=(s > 0))
               for c in range(_NCH)]
        for cp in cps:
            cp.wait()
    for c in range(_NCH):
        pltpu.sync_copy(acc_v.at[c], out_hbm.at[pl.ds(base + c * _CH, _CH)])


def _pool(text_h, table):
    mesh = plsc.VectorSubcoreMesh(core_axis_name="c", subcore_axis_name="s")
    return pl.kernel(
        _pool_body,
        out_type=jax.ShapeDtypeStruct((_BH, EMBED_), jnp.float32),
        mesh=mesh,
        scratch_types=[
            pltpu.VMEM((SEQ_, _BPW), jnp.int32),
            pltpu.VMEM((_NCH, _CH, EMBED_), jnp.float32),
            pltpu.SemaphoreType.DMA((_NCH,)),
        ],
    )(text_h, table)


def _split(x):
    hi = x.astype(jnp.bfloat16)
    lo = (x - hi.astype(jnp.float32)).astype(jnp.bfloat16)
    return hi, lo


def _outer_x3(x):
    hi, lo = _split(x)
    o = functools.partial(lax.dot_general,
                          dimension_numbers=(((0,), (0,)), ((), ())),
                          preferred_element_type=jnp.float32)
    hl = o(hi, lo)
    return o(hi, hi) + hl + hl.T


def _stats_body(x_ref, g_ref, s_ref):
    x = x_ref[...]
    g_ref[...] = _outer_x3(x)
    s_ref[...] = jnp.sum(x, axis=0, keepdims=True)


def _stats(psum_h):
    return pl.pallas_call(
        _stats_body,
        out_shape=[
            jax.ShapeDtypeStruct((EMBED_, EMBED_), jnp.float32),
            jax.ShapeDtypeStruct((1, EMBED_), jnp.float32),
        ],
    )(psum_h)


_dot = functools.partial(lax.dot_general, precision=lax.Precision.HIGHEST,
                         preferred_element_type=jnp.float32)


def _mm(a, b):
    return _dot(a, b, (((1,), (0,)), ((), ())))


def _outer(a, b):
    return _dot(a, b, (((0,), (0,)), ((), ())))


def _eye(n):
    r = lax.broadcasted_iota(jnp.int32, (n, n), 0)
    c = lax.broadcasted_iota(jnp.int32, (n, n), 1)
    return (r == c).astype(jnp.float32)


def _final_body(x0_ref, x1_ref, g0_ref, g1_ref, s0_ref, s1_ref,
                g1w_ref, be1_ref, w1_ref, b1_ref,
                g2w_ref, be2_ref, w2_ref, b2_ref,
                o_ref, a_s, k_s, d_s):
    i = pl.program_id(0)

    @pl.when(i == 0)
    def _fold():
        G = g0_ref[...] + g1_ref[...]
        ssum = s0_ref[...] + s1_ref[...]
        W1 = w1_ref[...]
        W2 = w2_ref[...]
        mu = ssum * (1.0 / (SEQ_ * BATCH_))
        Cov = G * (1.0 / (SEQ_ * SEQ_ * BATCH_)) - _outer(mu, mu)
        var1 = jnp.sum(Cov * _eye(EMBED_), axis=0, keepdims=True)
        a1 = g1w_ref[...] * lax.rsqrt(var1 + EPS_)
        c1 = be1_ref[...] - mu * a1
        CovA = Cov * _outer(a1, a1)
        T = _mm(CovA, W1)
        varh = jnp.sum(W1 * T, axis=0, keepdims=True)
        muh = _mm(be1_ref[...], W1) + b1_ref[...]
        a2 = g2w_ref[...] * lax.rsqrt(varh + EPS_)
        c2 = be2_ref[...] - muh * a2
        b1e = _mm(c1, W1) + b1_ref[...]
        a_s[...] = a1 * (1.0 / SEQ_)
        k_s[...] = _dot(W2, W1 * a2, (((0,), (1,)), ((), ())))
        d_s[...] = _mm(b1e * a2 + c2, W2) + b2_ref[...]

    def _emit(x):
        xa = x * a_s[...]
        cols = [jnp.sum(xa * k_s[j:j + 1, :], axis=1, keepdims=True)
                for j in range(OUT_)]
        o_ref[...] = jnp.concatenate(cols, axis=1) + d_s[...]

    @pl.when(i == 1)
    def _proj0():
        _emit(x0_ref[...])

    @pl.when(i == 2)
    def _proj1():
        _emit(x1_ref[...])


def _final(psum0, psum1, G0, G1, s0, s1, g1w, be1, W1, b1, g2w, be2, W2, b2):
    full = lambda shape: pl.BlockSpec(shape, lambda i: (0, 0))
    return pl.pallas_call(
        _final_body,
        grid=(1 + _NSL,),
        in_specs=[
            full((_BH, EMBED_)), full((_BH, EMBED_)),
            full((EMBED_, EMBED_)), full((EMBED_, EMBED_)),
            full((1, EMBED_)), full((1, EMBED_)),
            full((1, EMBED_)), full((1, EMBED_)),
            full((EMBED_, HIDDEN_)), full((1, HIDDEN_)),
            full((1, HIDDEN_)), full((1, HIDDEN_)),
            full((HIDDEN_, OUT_)), full((1, OUT_)),
        ],
        out_specs=pl.BlockSpec((_BH, OUT_),
                               lambda i: (jnp.maximum(i - 1, 0), 0)),
        out_shape=jax.ShapeDtypeStruct((BATCH_, OUT_), jnp.float32),
        scratch_shapes=[
            pltpu.VMEM((1, EMBED_), jnp.float32),
            pltpu.VMEM((OUT_, EMBED_), jnp.float32),
            pltpu.VMEM((1, OUT_), jnp.float32),
        ],
    )(psum0, psum1, G0, G1, s0, s1, g1w, be1, W1, b1, g2w, be2, W2, b2)


def kernel(text, label, embed_table, gamma1, beta1, W1, b1,
           gamma2, beta2, W2, b2):
    del label
    psum0 = _pool(text[:, :_BH], embed_table)
    psum1 = _pool(text[:, _BH:], embed_table)
    G0, s0 = _stats(psum0)
    G1, s1 = _stats(psum1)
    return _final(psum0, psum1, G0, G1, s0, s1,
                  gamma1.reshape(1, -1), beta1.reshape(1, -1), W1,
                  b1.reshape(1, -1), gamma2.reshape(1, -1),
                  beta2.reshape(1, -1), W2, b2.reshape(1, -1))


# speedup vs baseline: 1.1313x; 1.1313x over previous
import functools

import jax
import jax.numpy as jnp
from jax import lax
from jax.experimental import pallas as pl
from jax.experimental.pallas import tpu as pltpu
from jax.experimental.pallas import tpu_sc as plsc

VOCAB_ = 100000
EMBED_ = 128
HIDDEN_ = 1024
OUT_ = 2
SEQ_ = 20
BATCH_ = 16384
EPS_ = 1e-5

_NC = 2
_NS = 16
_NW = _NC * _NS
_BPW = BATCH_ // _NW
_CH = 64
_NCH = _BPW // _CH


def _pool_body(text_hbm, table_hbm, out_hbm, idx_v, acc_v, sems):
    wid = lax.axis_index("s") * _NC + lax.axis_index("c")
    base = wid * _BPW
    pltpu.sync_copy(text_hbm.at[:, pl.ds(base, _BPW)], idx_v)
    for s in range(SEQ_):
        cps = [pltpu.async_copy(
                   table_hbm.at[idx_v.at[s, pl.ds(c * _CH, _CH)]],
                   acc_v.at[c], sems.at[c], add=(s > 0))
               for c in range(_NCH)]
        for cp in cps:
            cp.wait()
    for c in range(_NCH):
        pltpu.sync_copy(acc_v.at[c], out_hbm.at[pl.ds(base + c * _CH, _CH)])


def _pool(text, table):
    mesh = plsc.VectorSubcoreMesh(core_axis_name="c", subcore_axis_name="s")
    return pl.kernel(
        _pool_body,
        out_type=jax.ShapeDtypeStruct((BATCH_, EMBED_), jnp.float32),
        mesh=mesh,
        scratch_types=[
            pltpu.VMEM((SEQ_, _BPW), jnp.int32),
            pltpu.VMEM((_NCH, _CH, EMBED_), jnp.float32),
            pltpu.SemaphoreType.DMA((_NCH,)),
        ],
    )(text, table)


_BB = 8192
_NB = BATCH_ // _BB


def _eye(n):
    r = lax.broadcasted_iota(jnp.int32, (n, n), 0)
    c = lax.broadcasted_iota(jnp.int32, (n, n), 1)
    return (r == c).astype(jnp.float32)


_dot = functools.partial(lax.dot_general, precision=lax.Precision.HIGHEST,
                         preferred_element_type=jnp.float32)


def _mm(a, b):
    return _dot(a, b, (((1,), (0,)), ((), ())))


def _outer(a, b):
    return _dot(a, b, (((0,), (0,)), ((), ())))


def _split(x):
    hi = x.astype(jnp.bfloat16)
    lo = (x - hi.astype(jnp.float32)).astype(jnp.bfloat16)
    return hi, lo


def _outer_x3(x):
    hi, lo = _split(x)
    o = functools.partial(lax.dot_general, dimension_numbers=(((0,), (0,)), ((), ())),
                          preferred_element_type=jnp.float32)
    hl = o(hi, lo)
    return o(hi, hi) + hl + hl.T


def _tc_body(xa_ref, xb_ref, g1_ref, be1_ref, w1_ref, b1_ref,
             g2_ref, be2_ref, w2_ref, b2_ref,
             o_ref, gacc, sacc):
    i = pl.program_id(0)

    @pl.when(i == 0)
    def _stats_a():
        x = xa_ref[...]
        gacc[...] = _outer_x3(x)
        sacc[...] = jnp.sum(x, axis=0, keepdims=True)

    @pl.when(i == 1)
    def _fold_proj():
        xb = xb_ref[...]
        G = gacc[...] + _outer_x3(xb)
        ssum = sacc[...] + jnp.sum(xb, axis=0, keepdims=True)
        W1 = w1_ref[...]
        W2 = w2_ref[...]
        mu = ssum * (1.0 / (SEQ_ * BATCH_))
        Cov = G * (1.0 / (SEQ_ * SEQ_ * BATCH_)) - _outer(mu, mu)
        var1 = jnp.sum(Cov * _eye(EMBED_), axis=0, keepdims=True)
        a1 = g1_ref[...] * lax.rsqrt(var1 + EPS_)
        c1 = be1_ref[...] - mu * a1
        CovA = Cov * _outer(a1, a1)
        T = _mm(CovA, W1)
        varh = jnp.sum(W1 * T, axis=0, keepdims=True)
        muh = _mm(be1_ref[...], W1) + b1_ref[...]
        a2 = g2_ref[...] * lax.rsqrt(varh + EPS_)
        c2 = be2_ref[...] - muh * a2
        b1e = _mm(c1, W1) + b1_ref[...]
        a = a1 * (1.0 / SEQ_)
        kT = _dot(W2, W1 * a2, (((0,), (1,)), ((), ())))
        d = _mm(b1e * a2 + c2, W2) + b2_ref[...]

        def _emit(x):
            xs = x * a
            cols = [jnp.sum(xs * kT[j:j + 1, :], axis=1, keepdims=True)
                    for j in range(OUT_)]
            return jnp.concatenate(cols, axis=1) + d

        o_ref[0:_BB, :] = _emit(xa_ref[...])
        o_ref[_BB:BATCH_, :] = _emit(xb)


def _tc_pipeline(psum, g1, be1, W1, b1, g2, be2, W2, b2):
    full = lambda shape: pl.BlockSpec(shape, lambda i: (0, 0))
    return pl.pallas_call(
        _tc_body,
        grid=(2,),
        in_specs=[
            pl.BlockSpec((_BB, EMBED_), lambda i: (0, 0)),
            pl.BlockSpec((_BB, EMBED_), lambda i: (1, 0)),
            full((1, EMBED_)), full((1, EMBED_)),
            full((EMBED_, HIDDEN_)), full((1, HIDDEN_)),
            full((1, HIDDEN_)), full((1, HIDDEN_)),
            full((HIDDEN_, OUT_)), full((1, OUT_)),
        ],
        out_specs=pl.BlockSpec((BATCH_, OUT_), lambda i: (0, 0)),
        out_shape=jax.ShapeDtypeStruct((BATCH_, OUT_), jnp.float32),
        scratch_shapes=[
            pltpu.VMEM((EMBED_, EMBED_), jnp.float32),
            pltpu.VMEM((1, EMBED_), jnp.float32),
        ],
    )(psum, psum, g1, be1, W1, b1, g2, be2, W2, b2)


def kernel(text, label, embed_table, gamma1, beta1, W1, b1,
           gamma2, beta2, W2, b2):
    del label
    psum = _pool(text, embed_table)
    return _tc_pipeline(psum,
                        gamma1.reshape(1, -1), beta1.reshape(1, -1), W1,
                        b1.reshape(1, -1), gamma2.reshape(1, -1),
                        beta2.reshape(1, -1), W2, b2.reshape(1, -1))


# speedup vs baseline: 1.1439x; 1.0112x over previous
import functools

import jax
import jax.numpy as jnp
from jax import lax
from jax.experimental import pallas as pl
from jax.experimental.pallas import tpu as pltpu
from jax.experimental.pallas import tpu_sc as plsc

VOCAB_ = 100000
EMBED_ = 128
HIDDEN_ = 1024
OUT_ = 2
SEQ_ = 20
BATCH_ = 16384
EPS_ = 1e-5

_NC = 2
_NS = 16
_NW = _NC * _NS
_BPW = BATCH_ // _NW
_CH = 64
_NCH = _BPW // _CH


def _pool_body(text_hbm, table_hbm, out_hbm, idx_v, acc_v, sems):
    wid = lax.axis_index("s") * _NC + lax.axis_index("c")
    base = wid * _BPW
    pltpu.sync_copy(text_hbm.at[:, pl.ds(base, _BPW)], idx_v)
    for s in range(SEQ_):
        cps = [pltpu.async_copy(
                   table_hbm.at[idx_v.at[s, pl.ds(c * _CH, _CH)]],
                   acc_v.at[c], sems.at[c], add=(s > 0))
               for c in range(_NCH)]
        for cp in cps:
            cp.wait()
    for c in range(_NCH):
        pltpu.sync_copy(acc_v.at[c], out_hbm.at[pl.ds(base + c * _CH, _CH)])


def _pool(text, table):
    mesh = plsc.VectorSubcoreMesh(core_axis_name="c", subcore_axis_name="s")
    return pl.kernel(
        _pool_body,
        out_type=jax.ShapeDtypeStruct((BATCH_, EMBED_), jnp.float32),
        mesh=mesh,
        scratch_types=[
            pltpu.VMEM((SEQ_, _BPW), jnp.int32),
            pltpu.VMEM((_NCH, _CH, EMBED_), jnp.float32),
            pltpu.SemaphoreType.DMA((_NCH,)),
        ],
    )(text, table)


_BB = 8192
_NB = BATCH_ // _BB


def _eye(n):
    r = lax.broadcasted_iota(jnp.int32, (n, n), 0)
    c = lax.broadcasted_iota(jnp.int32, (n, n), 1)
    return (r == c).astype(jnp.float32)


_dot = functools.partial(lax.dot_general, precision=lax.Precision.HIGHEST,
                         preferred_element_type=jnp.float32)


def _mm(a, b):
    return _dot(a, b, (((1,), (0,)), ((), ())))


def _outer(a, b):
    return _dot(a, b, (((0,), (0,)), ((), ())))


def _split(x):
    hi = x.astype(jnp.bfloat16)
    lo = (x - hi.astype(jnp.float32)).astype(jnp.bfloat16)
    return hi, lo


def _outer_x3(x):
    hi, lo = _split(x)
    o = functools.partial(lax.dot_general, dimension_numbers=(((0,), (0,)), ((), ())),
                          preferred_element_type=jnp.float32)
    hl = o(hi, lo)
    return o(hi, hi) + hl + hl.T


def _tc_body(x_ref, g1_ref, be1_ref, w1_ref, b1_ref,
             g2_ref, be2_ref, w2_ref, b2_ref,
             o_ref, gacc, sacc, a_s, k_s, d_s):
    i = pl.program_id(0)

    @pl.when(i < _NB)
    def _stats():
        x = x_ref[...]
        xtx = _outer_x3(x)
        cs = jnp.sum(x, axis=0, keepdims=True)

        @pl.when(i == 0)
        def _():
            gacc[...] = xtx
            sacc[...] = cs

        @pl.when(i > 0)
        def _():
            gacc[...] += xtx
            sacc[...] += cs

    @pl.when(i == _NB)
    def _fold():
        G = gacc[...]
        W1 = w1_ref[...]
        W2 = w2_ref[...]
        mu = sacc[...] * (1.0 / (SEQ_ * BATCH_))
        Cov = G * (1.0 / (SEQ_ * SEQ_ * BATCH_)) - _outer(mu, mu)
        var1 = jnp.sum(Cov * _eye(EMBED_), axis=0, keepdims=True)
        a1 = g1_ref[...] * lax.rsqrt(var1 + EPS_)
        c1 = be1_ref[...] - mu * a1
        CovA = Cov * _outer(a1, a1)
        T = _mm(CovA, W1)
        varh = jnp.sum(W1 * T, axis=0, keepdims=True)
        muh = _mm(be1_ref[...], W1) + b1_ref[...]
        a2 = g2_ref[...] * lax.rsqrt(varh + EPS_)
        c2 = be2_ref[...] - muh * a2
        b1e = _mm(c1, W1) + b1_ref[...]
        a_s[...] = a1 * (1.0 / SEQ_)
        k_s[...] = _dot(W2, W1 * a2, (((0,), (1,)), ((), ())))
        d_s[...] = _mm(b1e * a2 + c2, W2) + b2_ref[...]

    @pl.when(i > _NB)
    def _proj():
        xa = x_ref[...] * a_s[...]
        cols = [jnp.sum(xa * k_s[j:j + 1, :], axis=1, keepdims=True)
                for j in range(OUT_)]
        o_ref[...] = jnp.concatenate(cols, axis=1) + d_s[...]


def _tc_pipeline(psum, g1, be1, W1, b1, g2, be2, W2, b2):
    def x_map(i):
        return (jnp.where(i < _NB, i, jnp.maximum(i - _NB - 1, 0)), 0)

    def o_map(i):
        return (jnp.maximum(i - _NB - 1, 0), 0)

    full = lambda shape: pl.BlockSpec(shape, lambda i: (0, 0))
    return pl.pallas_call(
        _tc_body,
        grid=(2 * _NB + 1,),
        in_specs=[
            pl.BlockSpec((_BB, EMBED_), x_map),
            full((1, EMBED_)), full((1, EMBED_)),
            full((EMBED_, HIDDEN_)), full((1, HIDDEN_)),
            full((1, HIDDEN_)), full((1, HIDDEN_)),
            full((HIDDEN_, OUT_)), full((1, OUT_)),
        ],
        out_specs=pl.BlockSpec((_BB, OUT_), o_map),
        out_shape=jax.ShapeDtypeStruct((BATCH_, OUT_), jnp.float32),
        scratch_shapes=[
            pltpu.VMEM((EMBED_, EMBED_), jnp.float32),
            pltpu.VMEM((1, EMBED_), jnp.float32),
            pltpu.VMEM((1, EMBED_), jnp.float32),
            pltpu.VMEM((OUT_, EMBED_), jnp.float32),
            pltpu.VMEM((1, OUT_), jnp.float32),
        ],
    )(psum, g1, be1, W1, b1, g2, be2, W2, b2)


def kernel(text, label, embed_table, gamma1, beta1, W1, b1,
           gamma2, beta2, W2, b2):
    del label
    psum = _pool(text, embed_table)
    return _tc_pipeline(psum,
                        gamma1.reshape(1, -1), beta1.reshape(1, -1), W1,
                        b1.reshape(1, -1), gamma2.reshape(1, -1),
                        beta2.reshape(1, -1), W2, b2.reshape(1, -1))
